# baseline (speedup 1.0000x reference)
import functools

import jax
import jax.numpy as jnp
from jax import lax
from jax.experimental import pallas as pl
from jax.experimental.pallas import tpu as pltpu
from jax.experimental.pallas import tpu_sc as plsc

NC = 2
NS = 16
NW = NC * NS
CHUNK = 128
NSTAGE = 5
LANES = 16


def _make_spmm(n, d, epw, n_acc):
    cpw = epw // CHUNK
    cps = cpw // NSTAGE
    rpt = n_acc // NS
    cop = 128
    ncop = rpt // cop

    mesh = plsc.VectorSubcoreMesh(core_axis_name="c", subcore_axis_name="s")

    @functools.partial(
        pl.kernel,
        out_type=jax.ShapeDtypeStruct((NC, n_acc, d), jnp.float32),
        mesh=mesh,
        scratch_types=[
            pltpu.VMEM((cps, CHUNK), jnp.int32),
            pltpu.VMEM((cps, CHUNK), jnp.int32),
            pltpu.VMEM((cps, CHUNK), jnp.int32),
            pltpu.VMEM((cps, CHUNK), jnp.int32),
            pltpu.VMEM((cps, CHUNK), jnp.float32),
            pltpu.VMEM((cps, CHUNK), jnp.float32),
            pltpu.VMEM((CHUNK, d), jnp.float32),
            pltpu.VMEM((CHUNK, d), jnp.float32),
            pltpu.VMEM_SHARED((n_acc, d), jnp.float32),
            pltpu.SemaphoreType.DMA,
            pltpu.SemaphoreType.DMA,
            pltpu.SemaphoreType.DMA,
        ],
    )
    def spmm(x_hbm, cols_hbm, rows_hbm, w_hbm, out_hbm,
             cols_v0, cols_v1, rows_v0, rows_v1, w_v0, w_v1,
             buf0, buf1, acc, sem0, sem1, isem):
        c = lax.axis_index("c")
        s = lax.axis_index("s")
        wid = s * NC + c
        slots = ((cols_v0, rows_v0, w_v0), (cols_v1, rows_v1, w_v1))

        def idx_copies(h, slot):
            cv, rv, wv = slot
            sl = pl.ds(h * cps, cps)
            return (pltpu.make_async_copy(cols_hbm.at[wid, sl], cv, isem),
                    pltpu.make_async_copy(rows_hbm.at[wid, sl], rv, isem),
                    pltpu.make_async_copy(w_hbm.at[wid, sl], wv, isem))

        def load_idx(h, slot):
            for cp in idx_copies(h, slot):
                cp.start()

        def drain_idx(h, slot):
            for cp in idx_copies(h, slot):
                cp.wait()

        load_idx(0, slots[0])

        def zrow(r, carry):
            for j in range(d // LANES):
                buf0[r, pl.ds(j * LANES, LANES)] = jnp.zeros((LANES,), jnp.float32)
            return carry
        lax.fori_loop(0, cop, zrow, None)
        for i in range(ncop):
            r0 = s * rpt + i * cop
            pltpu.sync_copy(buf0.at[pl.ds(0, cop)], acc.at[pl.ds(r0, cop)])
        plsc.subcore_barrier()

        for h in range(NSTAGE):
            cols_v, rows_v, w_v = slots[h % 2]
            drain_idx(h, slots[h % 2])
            if h + 1 < NSTAGE:
                load_idx(h + 1, slots[(h + 1) % 2])

            def start_gather(ci, buf, sem):
                pltpu.async_copy(x_hbm.at[cols_v.at[ci]], buf, sem)

            def drain_gather(buf, sem):
                pltpu.make_async_copy(x_hbm.at[pl.ds(0, CHUNK)], buf, sem).wait()

            def scale(ci, buf):
                def grp(g, cc):
                    wv = w_v[ci, pl.ds(g * LANES, LANES)]
                    for l in range(LANES):
                        we = wv[l]
                        eidx = g * LANES + l
                        for j in range(d // LANES):
                            sl = pl.ds(j * LANES, LANES)
                            buf[eidx, sl] = buf[eidx, sl] * we
                    return cc
                lax.fori_loop(0, CHUNK // LANES, grp, None)

            start_gather(0, buf0, sem0)

            def pair_body(i2, carry):
                a = 2 * i2
                start_gather(a + 1, buf1, sem1)
                drain_gather(buf0, sem0)
                scale(a, buf0)
                pltpu.sync_copy(buf0, acc.at[rows_v.at[a]], add=True)

                @pl.when(a + 2 < cps)
                def _():
                    start_gather(a + 2, buf0, sem0)

                drain_gather(buf1, sem1)
                scale(a + 1, buf1)
                pltpu.sync_copy(buf1, acc.at[rows_v.at[a + 1]], add=True)
                return carry
            lax.fori_loop(0, cps // 2, pair_body, None)
        plsc.subcore_barrier()

        for i in range(ncop):
            r0 = s * rpt + i * cop
            pltpu.sync_copy(acc.at[pl.ds(r0, cop)], out_hbm.at[c, pl.ds(r0, cop)])

    return spmm


def _add_body(p0_ref, p1_ref, o_ref):
    o_ref[...] = p0_ref[0] + p1_ref[0]


def _final_body(x0_ref, x1_ref, q0_ref, q1_ref, wt_ref, b_ref, o_ref):
    x0 = x0_ref[...]
    x1 = x1_ref[...]
    x2 = 2.0 * (q0_ref[0] + q1_ref[0]) - x0
    acc = jnp.dot(x0, wt_ref[0], preferred_element_type=jnp.float32)
    acc += jnp.dot(x1, wt_ref[1], preferred_element_type=jnp.float32)
    acc += jnp.dot(x2, wt_ref[2], preferred_element_type=jnp.float32)
    o_ref[...] = acc + b_ref[...]


def kernel(x, edge_index, edge_weight, W, b):
    n, d = x.shape
    out_f = W.shape[0]
    k = W.shape[1] // d
    e = edge_index.shape[1]

    quant = NSTAGE * 8 * CHUNK
    epw = -(-e // (NW * quant)) * quant
    pad = epw * NW - e
    rows = edge_index[0].astype(jnp.int32)
    cols = edge_index[1].astype(jnp.int32)
    w = edge_weight.astype(jnp.float32)
    if pad:
        spread = (jnp.arange(pad, dtype=jnp.int32) * 31) % n
        rows = jnp.concatenate([rows, spread])
        cols = jnp.concatenate([cols, spread])
        w = jnp.concatenate([w, jnp.zeros((pad,), jnp.float32)])
    cpw = epw // CHUNK
    rows = rows.reshape(NW, cpw, CHUNK)
    cols = cols.reshape(NW, cpw, CHUNK)
    w = w.reshape(NW, cpw, CHUNK)

    n_acc = -(-n // (NS * 128)) * (NS * 128)
    spmm = _make_spmm(n, d, epw, n_acc)
    p = spmm(x, cols, rows, w)

    blk_a = n_acc // 8
    x1 = pl.pallas_call(
        _add_body,
        out_shape=jax.ShapeDtypeStruct((n_acc, d), jnp.float32),
        grid=(8,),
        in_specs=[pl.BlockSpec((1, blk_a, d), lambda i: (0, i, 0)),
                  pl.BlockSpec((1, blk_a, d), lambda i: (1, i, 0))],
        out_specs=pl.BlockSpec((blk_a, d), lambda i: (i, 0)),
    )(p, p)

    q = spmm(x1, cols, rows, w)

    blk = 1000 if n % 1000 == 0 else n
    grid = (n // blk,)

    wt = W.reshape(out_f, d, k).transpose(2, 1, 0)
    b2 = b.reshape(1, out_f)
    out = pl.pallas_call(
        _final_body,
        out_shape=jax.ShapeDtypeStruct((n, out_f), jnp.float32),
        grid=grid,
        in_specs=[pl.BlockSpec((blk, d), lambda i: (i, 0)),
                  pl.BlockSpec((blk, d), lambda i: (i, 0)),
                  pl.BlockSpec((1, blk, d), lambda i: (0, i, 0)),
                  pl.BlockSpec((1, blk, d), lambda i: (1, i, 0)),
                  pl.BlockSpec((k, d, out_f), lambda i: (0, 0, 0)),
                  pl.BlockSpec((1, out_f), lambda i: (0, 0))],
        out_specs=pl.BlockSpec((blk, out_f), lambda i: (i, 0)),
    )(x, x1, q, q, wt, b2)
    return out


# speedup vs baseline: 1.0345x; 1.0345x over previous
import functools

import jax
import jax.numpy as jnp
from jax import lax
from jax.experimental import pallas as pl
from jax.experimental.pallas import tpu as pltpu
from jax.experimental.pallas import tpu_sc as plsc

NC = 2
NS = 16
NW = NC * NS
CHUNK = 128
LANES = 16


def _make_spmm(n, d, epw, n_acc):
    cpw = epw // CHUNK
    rpt = n_acc // NS
    cop = 128
    ncop = rpt // cop

    mesh = plsc.VectorSubcoreMesh(core_axis_name="c", subcore_axis_name="s")

    @functools.partial(
        pl.kernel,
        out_type=jax.ShapeDtypeStruct((NC, n_acc, d), jnp.float32),
        mesh=mesh,
        scratch_types=[
            pltpu.VMEM((cpw // 2, CHUNK), jnp.int32),
            pltpu.VMEM((cpw // 2, CHUNK), jnp.int32),
            pltpu.VMEM((cpw // 2, CHUNK), jnp.float32),
            pltpu.VMEM((CHUNK, d), jnp.float32),
            pltpu.VMEM((CHUNK, d), jnp.float32),
            pltpu.VMEM_SHARED((n_acc, d), jnp.float32),
            pltpu.SemaphoreType.DMA,
            pltpu.SemaphoreType.DMA,
            pltpu.SemaphoreType.DMA,
            pltpu.SemaphoreType.DMA,
        ],
    )
    def spmm(x_hbm, cols_hbm, rows_hbm, w_hbm, out_hbm,
             cols_v, rows_v, w_v, buf0, buf1, acc, sem0, sem1, ssem0, ssem1):
        c = lax.axis_index("c")
        s = lax.axis_index("s")
        wid = s * NC + c
        cpw2 = cpw // 2

        def idx_copies(h):
            sl = pl.ds(h * cpw2, cpw2)
            return (pltpu.make_async_copy(cols_hbm.at[wid, sl], cols_v, ssem0),
                    pltpu.make_async_copy(rows_hbm.at[wid, sl], rows_v, ssem0),
                    pltpu.make_async_copy(w_hbm.at[wid, sl], w_v, ssem0))

        for cp in idx_copies(0):
            cp.start()

        def zrow(r, carry):
            for j in range(d // LANES):
                buf0[r, pl.ds(j * LANES, LANES)] = jnp.zeros((LANES,), jnp.float32)
            return carry
        lax.fori_loop(0, cop, zrow, None)
        for i in range(ncop):
            r0 = s * rpt + i * cop
            pltpu.sync_copy(buf0.at[pl.ds(0, cop)], acc.at[pl.ds(r0, cop)])
        plsc.subcore_barrier()

        def start_gather(ci, buf, sem):
            pltpu.async_copy(x_hbm.at[cols_v.at[ci]], buf, sem)

        def drain_gather(buf, sem):
            pltpu.make_async_copy(x_hbm.at[pl.ds(0, CHUNK)], buf, sem).wait()

        def scale(ci, buf):
            def grp(g, cc):
                wv = w_v[ci, pl.ds(g * LANES, LANES)]
                for l in range(LANES):
                    we = wv[l]
                    eidx = g * LANES + l
                    for j in range(d // LANES):
                        sl = pl.ds(j * LANES, LANES)
                        buf[eidx, sl] = buf[eidx, sl] * we
                return cc
            lax.fori_loop(0, CHUNK // LANES, grp, None)

        for h in range(2):
            if h == 0:
                for cp in idx_copies(0):
                    cp.wait()
            else:
                pltpu.sync_copy(cols_hbm.at[wid, pl.ds(h * cpw2, cpw2)], cols_v)
                pltpu.sync_copy(rows_hbm.at[wid, pl.ds(h * cpw2, cpw2)], rows_v)
                pltpu.sync_copy(w_hbm.at[wid, pl.ds(h * cpw2, cpw2)], w_v)
            start_gather(0, buf0, sem0)

            def pair_body(i2, carry):
                a = 2 * i2
                start_gather(a + 1, buf1, sem1)
                drain_gather(buf0, sem0)
                scale(a, buf0)
                pltpu.sync_copy(buf0, acc.at[rows_v.at[a]], add=True)

                @pl.when(a + 2 < cpw2)
                def _():
                    start_gather(a + 2, buf0, sem0)

                drain_gather(buf1, sem1)
                scale(a + 1, buf1)
                pltpu.sync_copy(buf1, acc.at[rows_v.at[a + 1]], add=True)
                return carry
            lax.fori_loop(0, cpw2 // 2, pair_body, None)
        plsc.subcore_barrier()

        for i in range(ncop):
            r0 = s * rpt + i * cop
            pltpu.sync_copy(acc.at[pl.ds(r0, cop)], out_hbm.at[c, pl.ds(r0, cop)])

    return spmm


def _add_body(p0_ref, p1_ref, o_ref):
    o_ref[...] = p0_ref[0] + p1_ref[0]


def _final_body(x0_ref, x1_ref, q0_ref, q1_ref, wt_ref, b_ref, o_ref):
    x0 = x0_ref[...]
    x1 = x1_ref[...]
    x2 = 2.0 * (q0_ref[0] + q1_ref[0]) - x0
    acc = jnp.dot(x0, wt_ref[0], preferred_element_type=jnp.float32)
    acc += jnp.dot(x1, wt_ref[1], preferred_element_type=jnp.float32)
    acc += jnp.dot(x2, wt_ref[2], preferred_element_type=jnp.float32)
    o_ref[...] = acc + b_ref[...]


def kernel(x, edge_index, edge_weight, W, b):
    n, d = x.shape
    out_f = W.shape[0]
    k = W.shape[1] // d
    e = edge_index.shape[1]

    epw = -(-e // (NW * 2 * CHUNK)) * 2 * CHUNK
    pad = epw * NW - e
    rows = edge_index[0].astype(jnp.int32)
    cols = edge_index[1].astype(jnp.int32)
    w = edge_weight.astype(jnp.float32)
    if pad:
        spread = (jnp.arange(pad, dtype=jnp.int32) * 31) % n
        rows = jnp.concatenate([rows, spread])
        cols = jnp.concatenate([cols, spread])
        w = jnp.concatenate([w, jnp.zeros((pad,), jnp.float32)])
    cpw = epw // CHUNK
    rows = rows.reshape(NW, cpw, CHUNK)
    cols = cols.reshape(NW, cpw, CHUNK)
    w = w.reshape(NW, cpw, CHUNK)

    n_acc = -(-n // (NS * 128)) * (NS * 128)
    spmm = _make_spmm(n, d, epw, n_acc)
    p = spmm(x, cols, rows, w)

    blk_a = n_acc // 8
    x1 = pl.pallas_call(
        _add_body,
        out_shape=jax.ShapeDtypeStruct((n_acc, d), jnp.float32),
        grid=(8,),
        in_specs=[pl.BlockSpec((1, blk_a, d), lambda i: (0, i, 0)),
                  pl.BlockSpec((1, blk_a, d), lambda i: (1, i, 0))],
        out_specs=pl.BlockSpec((blk_a, d), lambda i: (i, 0)),
    )(p, p)

    q = spmm(x1, cols, rows, w)

    blk = 1000 if n % 1000 == 0 else n
    grid = (n // blk,)

    wt = W.reshape(out_f, d, k).transpose(2, 1, 0)
    b2 = b.reshape(1, out_f)
    out = pl.pallas_call(
        _final_body,
        out_shape=jax.ShapeDtypeStruct((n, out_f), jnp.float32),
        grid=grid,
        in_specs=[pl.BlockSpec((blk, d), lambda i: (i, 0)),
                  pl.BlockSpec((blk, d), lambda i: (i, 0)),
                  pl.BlockSpec((1, blk, d), lambda i: (0, i, 0)),
                  pl.BlockSpec((1, blk, d), lambda i: (1, i, 0)),
                  pl.BlockSpec((k, d, out_f), lambda i: (0, 0, 0)),
                  pl.BlockSpec((1, out_f), lambda i: (0, 0))],
        out_specs=pl.BlockSpec((blk, out_f), lambda i: (i, 0)),
    )(x, x1, q, q, wt, b2)
    return out
